# baseline (speedup 1.0000x reference)
import functools

import jax
import jax.numpy as jnp
from jax import lax
from jax.experimental import pallas as pl
from jax.experimental.pallas import tpu as pltpu
from jax.experimental.pallas import tpu_sc as plsc

N_NODES = 10000
N_EDGES = 320000
D = 128

NC = 2
NS = 16
NW = NC * NS
EDGES_PER_TILE = N_EDGES // NW
CHUNK = 80
N_CHUNKS = EDGES_PER_TILE // CHUNK
ROWS_PER_TILE = 640
N_PAD = ROWS_PER_TILE * NS
L = 16


def _sc_segment_sum(edges, receivers, zeros_acc):
    mesh = plsc.VectorSubcoreMesh(core_axis_name="c", subcore_axis_name="s")

    @functools.partial(
        pl.kernel,
        out_type=[
            jax.ShapeDtypeStruct((NC * N_PAD, D), jnp.float32),
            jax.ShapeDtypeStruct((NC * N_PAD,), jnp.float32),
        ],
        mesh=mesh,
        scratch_types=[
            pltpu.VMEM((3, CHUNK), jnp.int32),
            pltpu.VMEM((3, CHUNK, D), jnp.float32),
            pltpu.VMEM((CHUNK,), jnp.float32),
            pltpu.VMEM((ROWS_PER_TILE,), jnp.float32),
            pltpu.VMEM_SHARED((N_PAD, D), jnp.float32),
            pltpu.VMEM_SHARED((N_PAD,), jnp.float32),
            pltpu.SemaphoreType.DMA,
            pltpu.SemaphoreType.DMA,
            pltpu.SemaphoreType.DMA,
            pltpu.SemaphoreType.DMA,
            pltpu.SemaphoreType.DMA,
            pltpu.SemaphoreType.DMA,
            pltpu.SemaphoreType.DMA,
            pltpu.SemaphoreType.DMA,
        ],
    )
    def k(edges_hbm, recv_hbm, zacc_hbm, out_sum, out_cnt,
          idx3, rows3, ones_v, cstage_v, acc_sh, cnt_sh,
          lsem0, lsem1, lsem2, ssem0, ssem1, ssem2, wsem0, wsem1):
        c = lax.axis_index("c")
        s = lax.axis_index("s")
        wid = c * NS + s
        r0 = s * ROWS_PER_TILE
        lsems = (lsem0, lsem1, lsem2)
        ssems = (ssem0, ssem1, ssem2)

        base = wid * EDGES_PER_TILE

        def start_load(b, i_off):
            pltpu.async_copy(recv_hbm.at[pl.ds(i_off, CHUNK)], idx3.at[b],
                             lsems[b])
            pltpu.async_copy(edges_hbm.at[pl.ds(i_off, CHUNK)],
                             rows3.at[b], lsems[b])

        def wait_load(b):
            pltpu.make_async_copy(recv_hbm.at[pl.ds(0, CHUNK)],
                                  idx3.at[b], lsems[b]).wait()
            pltpu.make_async_copy(edges_hbm.at[pl.ds(0, CHUNK)],
                                  rows3.at[b], lsems[b]).wait()

        def start_scatter(b):
            pltpu.async_copy(rows3.at[b], acc_sh.at[idx3.at[b]],
                             ssems[b], add=True)
            pltpu.async_copy(ones_v, cnt_sh.at[idx3.at[b]],
                             ssems[b], add=True)

        def wait_scatter(b):
            pltpu.make_async_copy(rows3.at[b], acc_sh.at[pl.ds(0, CHUNK)],
                                  ssems[b]).wait()
            pltpu.make_async_copy(ones_v, cnt_sh.at[pl.ds(0, CHUNK)],
                                  ssems[b]).wait()

        for b in range(2):
            start_load(b, base + b * CHUNK)

        for kk in range(CHUNK // L):
            ones_v[pl.ds(kk * L, L)] = jnp.ones((L,), jnp.float32)
        for kk in range(ROWS_PER_TILE // L):
            cstage_v[pl.ds(kk * L, L)] = jnp.zeros((L,), jnp.float32)

        pltpu.async_copy(zacc_hbm, acc_sh.at[pl.ds(r0, ROWS_PER_TILE)],
                         wsem0)
        pltpu.async_copy(cstage_v, cnt_sh.at[pl.ds(r0, ROWS_PER_TILE)],
                         wsem1)
        pltpu.make_async_copy(zacc_hbm, acc_sh.at[pl.ds(r0, ROWS_PER_TILE)],
                              wsem0).wait()
        pltpu.make_async_copy(cstage_v, cnt_sh.at[pl.ds(r0, ROWS_PER_TILE)],
                              wsem1).wait()

        plsc.subcore_barrier()

        @pl.loop(0, N_CHUNKS + 2, step=3)
        def _(g):
            for b in range(3):
                i = g + b

                @pl.when(i < N_CHUNKS)
                def _():
                    wait_load(b)
                    start_scatter(b)

                @pl.when(jnp.logical_and(i >= 1, i - 1 < N_CHUNKS))
                def _():
                    wait_scatter((b + 2) % 3)

                @pl.when(i + 2 < N_CHUNKS)
                def _():
                    start_load((b + 2) % 3, base + (i + 2) * CHUNK)

        plsc.subcore_barrier()

        o0 = c * N_PAD + r0
        pltpu.async_copy(acc_sh.at[pl.ds(r0, ROWS_PER_TILE)],
                         out_sum.at[pl.ds(o0, ROWS_PER_TILE)], wsem0)
        pltpu.async_copy(cnt_sh.at[pl.ds(r0, ROWS_PER_TILE)],
                         out_cnt.at[pl.ds(o0, ROWS_PER_TILE)], wsem1)
        pltpu.make_async_copy(acc_sh.at[pl.ds(r0, ROWS_PER_TILE)],
                              out_sum.at[pl.ds(o0, ROWS_PER_TILE)],
                              wsem0).wait()
        pltpu.make_async_copy(cnt_sh.at[pl.ds(r0, ROWS_PER_TILE)],
                              out_cnt.at[pl.ds(o0, ROWS_PER_TILE)],
                              wsem1).wait()

    return k(edges, receivers, zeros_acc)


BLK = 1000


def _tc_mlp_body(sum_ref, cnt_ref, w1, b1, w2, b2, w3, b3, out_ref):
    s = sum_ref[0] + sum_ref[1]
    cnt = cnt_ref[0] + cnt_ref[1]
    x = s / jnp.maximum(cnt, 1.0)
    h = jnp.dot(x, w1[...], preferred_element_type=jnp.float32) + b1[...]
    h = jnp.maximum(h, 0.0)
    h = jnp.dot(h, w2[...], preferred_element_type=jnp.float32) + b2[...]
    h = jnp.maximum(h, 0.0)
    out_ref[...] = jnp.dot(h, w3[...], preferred_element_type=jnp.float32) + b3[...]


def _tc_mlp(sums, cnts, W1, b1, W2, b2, W3, b3):
    grid = (N_NODES // BLK,)
    full = lambda shape: pl.BlockSpec(shape, lambda i: (0,) * len(shape))
    return pl.pallas_call(
        _tc_mlp_body,
        grid=grid,
        in_specs=[
            pl.BlockSpec((NC, BLK, D), lambda i: (0, i, 0)),
            pl.BlockSpec((NC, BLK, 1), lambda i: (0, i, 0)),
            full((D, 128)), full((1, 128)),
            full((128, 64)), full((1, 64)),
            full((64, D)), full((1, D)),
        ],
        out_specs=pl.BlockSpec((BLK, D), lambda i: (i, 0)),
        out_shape=jax.ShapeDtypeStruct((N_NODES, D), jnp.float32),
    )(sums, cnts, W1, b1, W2, b2, W3, b3)


def kernel(nodes, edges, senders, receivers, W1, b1, W2, b2, W3, b3):
    del nodes, senders
    zeros_acc = jnp.zeros((ROWS_PER_TILE, D), jnp.float32)
    sums, cnts = _sc_segment_sum(edges, receivers, zeros_acc)
    sums = sums.reshape(NC, N_PAD, D)
    cnts = cnts.reshape(NC, N_PAD, 1)
    return _tc_mlp(sums, cnts, W1, b1.reshape(1, -1), W2, b2.reshape(1, -1),
                   W3, b3.reshape(1, -1))


# speedup vs baseline: 1.0759x; 1.0759x over previous
import functools

import jax
import jax.numpy as jnp
from jax import lax
from jax.experimental import pallas as pl
from jax.experimental.pallas import tpu as pltpu
from jax.experimental.pallas import tpu_sc as plsc

N_NODES = 10000
N_EDGES = 320000
D = 128

NC = 2
NS = 16
NW = NC * NS
EDGES_PER_TILE = N_EDGES // NW
CHUNK = 80
N_CHUNKS = EDGES_PER_TILE // CHUNK
ROWS_PER_TILE = 640
N_PAD = ROWS_PER_TILE * NS
STG = CHUNK
N_STG = ROWS_PER_TILE // STG
L = 16


def _sc_segment_sum(edges, receivers, zeros_acc):
    mesh = plsc.VectorSubcoreMesh(core_axis_name="c", subcore_axis_name="s")

    @functools.partial(
        pl.kernel,
        out_type=[
            jax.ShapeDtypeStruct((NC * N_PAD, D), jnp.float32),
            jax.ShapeDtypeStruct((NC * N_PAD,), jnp.float32),
        ],
        mesh=mesh,
        scratch_types=[
            pltpu.VMEM((3, CHUNK), jnp.int32),
            pltpu.VMEM((3, CHUNK, D), jnp.float32),
            pltpu.VMEM((CHUNK,), jnp.float32),
            pltpu.VMEM((ROWS_PER_TILE,), jnp.float32),
            pltpu.VMEM_SHARED((N_PAD, D), jnp.float32),
            pltpu.VMEM_SHARED((N_PAD,), jnp.float32),
            pltpu.SemaphoreType.DMA,
            pltpu.SemaphoreType.DMA,
            pltpu.SemaphoreType.DMA,
            pltpu.SemaphoreType.DMA,
            pltpu.SemaphoreType.DMA,
            pltpu.SemaphoreType.DMA,
            pltpu.SemaphoreType.DMA,
            pltpu.SemaphoreType.DMA,
        ],
    )
    def k(edges_hbm, recv_hbm, zacc_hbm, out_sum, out_cnt,
          idx3, rows3, ones_v, cstage_v, acc_sh, cnt_sh,
          lsem0, lsem1, lsem2, ssem0, ssem1, ssem2, wsem0, wsem1):
        c = lax.axis_index("c")
        s = lax.axis_index("s")
        wid = c * NS + s
        r0 = s * ROWS_PER_TILE
        lsems = (lsem0, lsem1, lsem2)
        ssems = (ssem0, ssem1, ssem2)
        wsems = (wsem0, wsem1)

        base = wid * EDGES_PER_TILE

        def start_load(b, i_off):
            pltpu.async_copy(recv_hbm.at[pl.ds(i_off, CHUNK)], idx3.at[b],
                             lsems[b])
            pltpu.async_copy(edges_hbm.at[pl.ds(i_off, CHUNK)],
                             rows3.at[b], lsems[b])

        def wait_load(b):
            pltpu.make_async_copy(recv_hbm.at[pl.ds(0, CHUNK)],
                                  idx3.at[b], lsems[b]).wait()
            pltpu.make_async_copy(edges_hbm.at[pl.ds(0, CHUNK)],
                                  rows3.at[b], lsems[b]).wait()

        def start_scatter(b):
            pltpu.async_copy(rows3.at[b], acc_sh.at[idx3.at[b]],
                             ssems[b], add=True)
            pltpu.async_copy(ones_v, cnt_sh.at[idx3.at[b]],
                             ssems[b], add=True)

        def wait_scatter(b):
            pltpu.make_async_copy(rows3.at[b], acc_sh.at[pl.ds(0, CHUNK)],
                                  ssems[b]).wait()
            pltpu.make_async_copy(ones_v, cnt_sh.at[pl.ds(0, CHUNK)],
                                  ssems[b]).wait()

        for b in range(2):
            start_load(b, base + b * CHUNK)

        for kk in range(CHUNK // L):
            ones_v[pl.ds(kk * L, L)] = jnp.ones((L,), jnp.float32)
        for kk in range(ROWS_PER_TILE // L):
            cstage_v[pl.ds(kk * L, L)] = jnp.zeros((L,), jnp.float32)

        pltpu.sync_copy(zacc_hbm, rows3.at[2])
        for j in range(N_STG):
            pltpu.async_copy(rows3.at[2],
                             acc_sh.at[pl.ds(r0 + j * STG, STG)], wsem0)
        pltpu.async_copy(cstage_v, cnt_sh.at[pl.ds(r0, ROWS_PER_TILE)],
                         wsem0)
        for j in range(N_STG):
            pltpu.make_async_copy(rows3.at[2],
                                  acc_sh.at[pl.ds(r0, STG)], wsem0).wait()
        pltpu.make_async_copy(cstage_v, cnt_sh.at[pl.ds(r0, ROWS_PER_TILE)],
                              wsem0).wait()

        plsc.subcore_barrier()

        @pl.loop(0, N_CHUNKS + 2, step=3)
        def _(g):
            for b in range(3):
                i = g + b

                @pl.when(i < N_CHUNKS)
                def _():
                    wait_load(b)
                    start_scatter(b)

                @pl.when(jnp.logical_and(i >= 1, i - 1 < N_CHUNKS))
                def _():
                    wait_scatter((b + 2) % 3)

                @pl.when(i + 2 < N_CHUNKS)
                def _():
                    start_load((b + 2) % 3, base + (i + 2) * CHUNK)

        plsc.subcore_barrier()

        o0 = c * N_PAD + r0
        for j in range(N_STG):
            b = j % 2
            if j >= 2:
                pltpu.make_async_copy(rows3.at[b],
                                      out_sum.at[pl.ds(o0, STG)],
                                      wsems[b]).wait()
            pltpu.sync_copy(acc_sh.at[pl.ds(r0 + j * STG, STG)], rows3.at[b])
            pltpu.async_copy(rows3.at[b],
                             out_sum.at[pl.ds(o0 + j * STG, STG)], wsems[b])
        pltpu.sync_copy(cnt_sh.at[pl.ds(r0, ROWS_PER_TILE)], cstage_v)
        pltpu.sync_copy(cstage_v, out_cnt.at[pl.ds(o0, ROWS_PER_TILE)])
        for b in range(2):
            pltpu.make_async_copy(rows3.at[b], out_sum.at[pl.ds(o0, STG)],
                                  wsems[b]).wait()

    return k(edges, receivers, zeros_acc)


BLK = 1024


def _tc_mlp_body(sum_ref, cnt_ref, w1, b1, w2, b2, w3, b3, out_ref):
    s = sum_ref[0] + sum_ref[1]
    ones_w = jnp.ones((NC, 1), jnp.float32)
    cnt = lax.dot_general(cnt_ref[...], ones_w, (((0,), (0,)), ((), ())),
                          preferred_element_type=jnp.float32)
    x = s / jnp.maximum(cnt, 1.0)
    h = jnp.dot(x, w1[...], preferred_element_type=jnp.float32) + b1[...]
    h = jnp.maximum(h, 0.0)
    h = jnp.dot(h, w2[...], preferred_element_type=jnp.float32) + b2[...]
    h = jnp.maximum(h, 0.0)
    out_ref[...] = jnp.dot(h, w3[...], preferred_element_type=jnp.float32) + b3[...]


def _tc_mlp(sums, cnts, W1, b1, W2, b2, W3, b3):
    grid = (N_PAD // BLK,)
    full = lambda shape: pl.BlockSpec(shape, lambda i: (0,) * len(shape))
    return pl.pallas_call(
        _tc_mlp_body,
        grid=grid,
        in_specs=[
            pl.BlockSpec((NC, BLK, D), lambda i: (0, i, 0)),
            pl.BlockSpec((NC, BLK), lambda i: (0, i)),
            full((D, 128)), full((1, 128)),
            full((128, 64)), full((1, 64)),
            full((64, D)), full((1, D)),
        ],
        out_specs=pl.BlockSpec((BLK, D), lambda i: (i, 0)),
        out_shape=jax.ShapeDtypeStruct((N_PAD, D), jnp.float32),
    )(sums, cnts, W1, b1, W2, b2, W3, b3)


def kernel(nodes, edges, senders, receivers, W1, b1, W2, b2, W3, b3):
    del nodes, senders
    zeros_acc = jnp.zeros((CHUNK, D), jnp.float32)
    sums, cnts = _sc_segment_sum(edges, receivers, zeros_acc)
    sums = sums.reshape(NC, N_PAD, D)
    cnts = cnts.reshape(NC, N_PAD)
    out = _tc_mlp(sums, cnts, W1, b1.reshape(1, -1), W2, b2.reshape(1, -1),
                  W3, b3.reshape(1, -1))
    return out[:N_NODES]


# speedup vs baseline: 1.0851x; 1.0085x over previous
import functools

import jax
import jax.numpy as jnp
from jax import lax
from jax.experimental import pallas as pl
from jax.experimental.pallas import tpu as pltpu
from jax.experimental.pallas import tpu_sc as plsc

N_NODES = 10000
N_EDGES = 320000
D = 128

NC = 2
NS = 16
NW = NC * NS
EDGES_PER_TILE = N_EDGES // NW
CHUNK = 80
N_CHUNKS = EDGES_PER_TILE // CHUNK
ROWS_PER_TILE = 640
N_PAD = ROWS_PER_TILE * NS
STG = CHUNK
N_STG = ROWS_PER_TILE // STG
L = 16


def _sc_segment_sum(edges, receivers, zeros_acc):
    mesh = plsc.VectorSubcoreMesh(core_axis_name="c", subcore_axis_name="s")

    @functools.partial(
        pl.kernel,
        out_type=[
            jax.ShapeDtypeStruct((NC * N_PAD, D), jnp.float32),
            jax.ShapeDtypeStruct((NC * N_PAD,), jnp.float32),
        ],
        mesh=mesh,
        scratch_types=[
            pltpu.VMEM((3, CHUNK), jnp.int32),
            pltpu.VMEM((3, CHUNK, D), jnp.float32),
            pltpu.VMEM((CHUNK,), jnp.float32),
            pltpu.VMEM((ROWS_PER_TILE,), jnp.float32),
            pltpu.VMEM_SHARED((N_PAD, D), jnp.float32),
            pltpu.VMEM_SHARED((N_PAD,), jnp.float32),
            pltpu.SemaphoreType.DMA,
            pltpu.SemaphoreType.DMA,
            pltpu.SemaphoreType.DMA,
            pltpu.SemaphoreType.DMA,
            pltpu.SemaphoreType.DMA,
            pltpu.SemaphoreType.DMA,
            pltpu.SemaphoreType.DMA,
            pltpu.SemaphoreType.DMA,
        ],
    )
    def k(edges_hbm, recv_hbm, zacc_hbm, out_sum, out_cnt,
          idx3, rows3, ones_v, cstage_v, acc_sh, cnt_sh,
          lsem0, lsem1, lsem2, ssem0, ssem1, ssem2, wsem0, wsem1):
        c = lax.axis_index("c")
        s = lax.axis_index("s")
        wid = c * NS + s
        r0 = s * ROWS_PER_TILE
        lsems = (lsem0, lsem1, lsem2)
        ssems = (ssem0, ssem1, ssem2)
        wsems = (wsem0, wsem1)

        for kk in range(CHUNK // L):
            ones_v[pl.ds(kk * L, L)] = jnp.ones((L,), jnp.float32)
        for kk in range(ROWS_PER_TILE // L):
            cstage_v[pl.ds(kk * L, L)] = jnp.zeros((L,), jnp.float32)

        pltpu.sync_copy(zacc_hbm, rows3.at[0])
        for j in range(N_STG):
            pltpu.async_copy(rows3.at[0],
                             acc_sh.at[pl.ds(r0 + j * STG, STG)], wsem0)
        pltpu.async_copy(cstage_v, cnt_sh.at[pl.ds(r0, ROWS_PER_TILE)],
                         wsem0)
        for j in range(N_STG):
            pltpu.make_async_copy(rows3.at[0],
                                  acc_sh.at[pl.ds(r0, STG)], wsem0).wait()
        pltpu.make_async_copy(cstage_v, cnt_sh.at[pl.ds(r0, ROWS_PER_TILE)],
                              wsem0).wait()

        base = wid * EDGES_PER_TILE

        def start_load(b, i_off):
            pltpu.async_copy(recv_hbm.at[pl.ds(i_off, CHUNK)], idx3.at[b],
                             lsems[b])
            pltpu.async_copy(edges_hbm.at[pl.ds(i_off, CHUNK)],
                             rows3.at[b], lsems[b])

        def wait_load(b):
            pltpu.make_async_copy(recv_hbm.at[pl.ds(0, CHUNK)],
                                  idx3.at[b], lsems[b]).wait()
            pltpu.make_async_copy(edges_hbm.at[pl.ds(0, CHUNK)],
                                  rows3.at[b], lsems[b]).wait()

        def start_scatter(b):
            pltpu.async_copy(rows3.at[b], acc_sh.at[idx3.at[b]],
                             ssems[b], add=True)
            pltpu.async_copy(ones_v, cnt_sh.at[idx3.at[b]],
                             ssems[b], add=True)

        def wait_scatter(b):
            pltpu.make_async_copy(rows3.at[b], acc_sh.at[pl.ds(0, CHUNK)],
                                  ssems[b]).wait()
            pltpu.make_async_copy(ones_v, cnt_sh.at[pl.ds(0, CHUNK)],
                                  ssems[b]).wait()

        for b in range(2):
            start_load(b, base + b * CHUNK)

        plsc.subcore_barrier()

        @pl.loop(0, N_CHUNKS + 2, step=3)
        def _(g):
            for b in range(3):
                i = g + b

                @pl.when(i < N_CHUNKS)
                def _():
                    wait_load(b)
                    start_scatter(b)

                @pl.when(jnp.logical_and(i >= 1, i - 1 < N_CHUNKS))
                def _():
                    wait_scatter((b + 2) % 3)

                @pl.when(i + 2 < N_CHUNKS)
                def _():
                    start_load((b + 2) % 3, base + (i + 2) * CHUNK)

        plsc.subcore_barrier()

        o0 = c * N_PAD + r0
        for j in range(N_STG):
            b = j % 2
            if j >= 2:
                pltpu.make_async_copy(rows3.at[b],
                                      out_sum.at[pl.ds(o0, STG)],
                                      wsems[b]).wait()
            pltpu.sync_copy(acc_sh.at[pl.ds(r0 + j * STG, STG)], rows3.at[b])
            pltpu.async_copy(rows3.at[b],
                             out_sum.at[pl.ds(o0 + j * STG, STG)], wsems[b])
        pltpu.sync_copy(cnt_sh.at[pl.ds(r0, ROWS_PER_TILE)], cstage_v)
        pltpu.sync_copy(cstage_v, out_cnt.at[pl.ds(o0, ROWS_PER_TILE)])
        for b in range(2):
            pltpu.make_async_copy(rows3.at[b], out_sum.at[pl.ds(o0, STG)],
                                  wsems[b]).wait()

    return k(edges, receivers, zeros_acc)


BLK = 1024


def _tc_mlp_body(sum_ref, cnt_ref, w1, b1, w2, b2, w3, b3, out_ref):
    s = sum_ref[0] + sum_ref[1]
    ones_w = jnp.ones((NC, 1), jnp.float32)
    cnt = lax.dot_general(cnt_ref[...], ones_w, (((0,), (0,)), ((), ())),
                          preferred_element_type=jnp.float32)
    x = s / jnp.maximum(cnt, 1.0)
    h = jnp.dot(x, w1[...], preferred_element_type=jnp.float32) + b1[...]
    h = jnp.maximum(h, 0.0)
    h = jnp.dot(h, w2[...], preferred_element_type=jnp.float32) + b2[...]
    h = jnp.maximum(h, 0.0)
    out_ref[...] = jnp.dot(h, w3[...], preferred_element_type=jnp.float32) + b3[...]


def _tc_mlp(sums, cnts, W1, b1, W2, b2, W3, b3):
    grid = (N_PAD // BLK,)
    full = lambda shape: pl.BlockSpec(shape, lambda i: (0,) * len(shape))
    return pl.pallas_call(
        _tc_mlp_body,
        grid=grid,
        in_specs=[
            pl.BlockSpec((NC, BLK, D), lambda i: (0, i, 0)),
            pl.BlockSpec((NC, BLK), lambda i: (0, i)),
            full((D, 128)), full((1, 128)),
            full((128, 64)), full((1, 64)),
            full((64, D)), full((1, D)),
        ],
        out_specs=pl.BlockSpec((BLK, D), lambda i: (i, 0)),
        out_shape=jax.ShapeDtypeStruct((N_PAD, D), jnp.float32),
    )(sums, cnts, W1, b1, W2, b2, W3, b3)


def kernel(nodes, edges, senders, receivers, W1, b1, W2, b2, W3, b3):
    del nodes, senders
    zeros_acc = jnp.zeros((CHUNK, D), jnp.float32)
    sums, cnts = _sc_segment_sum(edges, receivers, zeros_acc)
    sums = sums.reshape(NC, N_PAD, D)
    cnts = cnts.reshape(NC, N_PAD)
    out = _tc_mlp(sums, cnts, W1, b1.reshape(1, -1), W2, b2.reshape(1, -1),
                  W3, b3.reshape(1, -1))
    return out[:N_NODES]
